# baseline (speedup 1.0000x reference)
import functools

import jax
import jax.numpy as jnp
from jax import lax
from jax.experimental import pallas as pl
from jax.experimental.pallas import tpu as pltpu
from jax.experimental.pallas import tpu_sc as plsc

C = 96
H = W = 224
LH = 75
L = LH * LH
LP = 5632
CK = 864
CKP = 896

QT = 256
NQT = LP // QT

NPIX = H * W
PT = 6272

WPAD = 232
HHP = 226
C2T = 6560
C2N = 8
C2FLAT = C2T * (C2N + 1)
OFFS = tuple(WPAD * dy + dx for dy in range(3) for dx in range(3))

_BIG_I = 1 << 30


def _corr_body(q_ref, kn_ref, id2_ref):
    qn = q_ref[...]
    s = jax.lax.dot_general(
        qn, kn_ref[...], (((1,), (0,)), ((), ())),
        preferred_element_type=jnp.float32)
    col = lax.broadcasted_iota(jnp.int32, s.shape, 1)
    s = jnp.where(col < L, s, -2.0)
    m1 = jnp.max(s, axis=1, keepdims=True)
    i1 = jnp.min(jnp.where(s == m1, col, _BIG_I), axis=1, keepdims=True)
    s2 = jnp.where(col == i1, -3.0, s)
    m2 = jnp.max(s2, axis=1, keepdims=True)
    i2 = jnp.min(jnp.where(s2 == m2, col, _BIG_I), axis=1, keepdims=True)
    id2_ref[0, 0, :] = i2[:, 0]


def _top2_indices(qraw, kn):
    id2 = pl.pallas_call(
        _corr_body,
        grid=(NQT,),
        in_specs=[
            pl.BlockSpec((QT, CKP), lambda i: (i, 0)),
            pl.BlockSpec((CKP, LP), lambda i: (0, 0)),
        ],
        out_specs=pl.BlockSpec((1, 1, QT), lambda i: (i, 0, 0)),
        out_shape=jax.ShapeDtypeStruct((NQT, 1, QT), jnp.int32),
    )(qraw, kn)
    return id2.reshape(LP)


def _sc_gather(table, idx):
    try:
        info = plsc.get_sparse_core_info()
        nc, ns = info.num_cores, info.num_subcores
    except Exception:
        nc, ns = 2, 16
    nw = nc * ns
    bpw = LP // nw
    nch = 2
    gr = bpw // nch

    @functools.partial(
        pl.kernel,
        out_type=jax.ShapeDtypeStruct((LP, CKP), jnp.float32),
        mesh=plsc.VectorSubcoreMesh(core_axis_name="c", subcore_axis_name="s"),
        scratch_types=[
            pltpu.VMEM((gr,), jnp.int32),
            pltpu.VMEM((gr, CKP), jnp.float32),
            pltpu.SemaphoreType.DMA,
        ],
    )
    def gather_kernel(table_hbm, idx_hbm, out_hbm, idx_v, rows_v, sem):
        wid = lax.axis_index("s") * nc + lax.axis_index("c")
        base = wid * bpw
        for ch in range(nch):
            b = base + ch * gr
            pltpu.sync_copy(idx_hbm.at[pl.ds(b, gr)], idx_v)
            pltpu.async_copy(table_hbm.at[idx_v], rows_v, sem).wait()
            pltpu.sync_copy(rows_v, out_hbm.at[pl.ds(b, gr)])

    return gather_kernel(table, idx)


def _c1_body(x_ref, t_ref, wa_ref, wb_ref, b_ref, a_ref, o_ref):
    y = jax.lax.dot_general(
        x_ref[...], wa_ref[...], (((1,), (0,)), ((), ())),
        preferred_element_type=jnp.float32)
    y = y + jax.lax.dot_general(
        t_ref[...], wb_ref[...], (((1,), (0,)), ((), ())),
        preferred_element_type=jnp.float32)
    y = y + b_ref[...]
    a = a_ref[0, 0]
    o_ref[...] = jnp.where(y >= 0, y, a * y)


def _conv1(x_pix, t_pix, wa, wb, b1, a):
    return pl.pallas_call(
        _c1_body,
        grid=(NPIX // PT,),
        in_specs=[
            pl.BlockSpec((PT, C), lambda i: (i, 0)),
            pl.BlockSpec((PT, C), lambda i: (i, 0)),
            pl.BlockSpec((C, C), lambda i: (0, 0)),
            pl.BlockSpec((C, C), lambda i: (0, 0)),
            pl.BlockSpec((1, C), lambda i: (0, 0)),
            pl.BlockSpec((1, 1), lambda i: (0, 0)),
        ],
        out_specs=pl.BlockSpec((PT, C), lambda i: (i, 0)),
        out_shape=jax.ShapeDtypeStruct((NPIX, C), jnp.float32),
    )(x_pix, t_pix, wa, wb, b1, a)


def _c2_body(ha_ref, hb_ref, w_ref, b_ref, y_ref):
    acc = jnp.zeros((C2T, C), jnp.float32)
    for si, off in enumerate(OFFS):
        if off == 0:
            hs = ha_ref[...]
        else:
            hs = jnp.concatenate([ha_ref[off:, :], hb_ref[:off, :]], axis=0)
        acc = acc + jax.lax.dot_general(
            hs, w_ref[si], (((1,), (0,)), ((), ())),
            preferred_element_type=jnp.float32)
    y_ref[...] = acc + b_ref[...]


def _conv2(hflat, w2s, b2):
    return pl.pallas_call(
        _c2_body,
        grid=(C2N,),
        in_specs=[
            pl.BlockSpec((C2T, C), lambda i: (i, 0)),
            pl.BlockSpec((C2T, C), lambda i: (i + 1, 0)),
            pl.BlockSpec((9, C, C), lambda i: (0, 0, 0)),
            pl.BlockSpec((1, C), lambda i: (0, 0)),
        ],
        out_specs=pl.BlockSpec((C2T, C), lambda i: (i, 0)),
        out_shape=jax.ShapeDtypeStruct((C2N * C2T, C), jnp.float32),
    )(hflat, hflat, w2s, b2)


def kernel(x8, x4, W1, b1, prelu_a, W2, b2):
    f32 = jnp.float32
    x4i = x4[0].astype(f32)
    x8i = x8[0].astype(f32)

    x4p = jnp.pad(x4i, ((0, 0), (1, 1), (1, 1)))[:, :225, :225]
    x8p = jnp.pad(x8i, ((0, 0), (1, 1), (1, 1)))[:, :225, :225]

    k5 = x4p.reshape(C, LH, 3, LH, 3)
    kraw = k5.transpose(0, 2, 4, 1, 3).reshape(1, CK, L)
    knn = jnp.sqrt(jnp.sum(kraw * kraw, axis=1, keepdims=True))
    kn = (kraw / jnp.maximum(knn, 1e-12))[0]
    kn = jnp.pad(kn, ((0, CKP - CK), (0, LP - L)))

    q5 = x8p.reshape(C, LH, 3, LH, 3)
    qraw = q5.transpose(1, 3, 0, 2, 4).reshape(1, L, CK)
    qnn = jnp.sqrt(jnp.sum(qraw * qraw, axis=2, keepdims=True))
    qn = (qraw / jnp.maximum(qnn, 1e-12))[0]
    qn = jnp.pad(qn, ((0, LP - L), (0, CKP - CK)))

    id2 = _top2_indices(qn, kn)

    xt = x4p.transpose(1, 2, 0)
    table = xt.reshape(LH, 3, LH, 3, C).transpose(0, 2, 1, 3, 4).reshape(L, CK)
    table = jnp.pad(table, ((0, LP - L), (0, CKP - CK)))

    tr = _sc_gather(table, id2)

    t225 = tr[:L, :CK].reshape(LH, LH, 3, 3, C).transpose(0, 2, 1, 3, 4)
    t225 = t225.reshape(225, 225, C)
    t_pix = t225[1:225, 1:225, :].reshape(NPIX, C)
    x4_pix = x4i.transpose(1, 2, 0).reshape(NPIX, C)

    w1t = W1[:, :, 0, 0].T.astype(f32)
    h_pix = _conv1(x4_pix, t_pix, w1t[:C], w1t[C:], b1.reshape(1, C),
                   prelu_a.reshape(1, 1))

    h_img = h_pix.reshape(H, W, C)
    hp = jnp.pad(h_img, ((1, 1), (1, WPAD - W - 1), (0, 0)))
    hflat = hp.reshape(HHP * WPAD, C)
    hflat = jnp.pad(hflat, ((0, C2FLAT - HHP * WPAD), (0, 0)))

    w2s = W2.transpose(2, 3, 1, 0).reshape(9, C, C)
    y = _conv2(hflat, w2s, b2.reshape(1, C))

    out = y[: HHP * WPAD].reshape(HHP, WPAD, C)[:H, :W, :]
    return out.transpose(2, 0, 1)[None].astype(x8.dtype)


# speedup vs baseline: 1.0309x; 1.0309x over previous
import functools

import jax
import jax.numpy as jnp
from jax import lax
from jax.experimental import pallas as pl
from jax.experimental.pallas import tpu as pltpu
from jax.experimental.pallas import tpu_sc as plsc

C = 96
H = W = 224
LH = 75
L = LH * LH
LP = 5632
CK = 864
CKP = 896

QT = 256
NQT = LP // QT

NPIX = H * W

WPAD = 232
C1RB = 8
C1N = 35
C1BLK = C1RB * WPAD
HROWS = C1N * C1BLK
C2T = 6496
C2N = 8
C2BASE = 7 * WPAD
OFFS = tuple(C2BASE + WPAD * dy + dx for dy in range(3) for dx in range(3))

_BIG_I = 1 << 30


def _corr_body(q_ref, kn_ref, id2_ref):
    qn = q_ref[...]
    s = jax.lax.dot_general(
        qn, kn_ref[...], (((1,), (0,)), ((), ())),
        preferred_element_type=jnp.float32)
    col = lax.broadcasted_iota(jnp.int32, s.shape, 1)
    s = jnp.where(col < L, s, -2.0)
    m1 = jnp.max(s, axis=1, keepdims=True)
    i1 = jnp.min(jnp.where(s == m1, col, _BIG_I), axis=1, keepdims=True)
    s2 = jnp.where(col == i1, -3.0, s)
    m2 = jnp.max(s2, axis=1, keepdims=True)
    i2 = jnp.min(jnp.where(s2 == m2, col, _BIG_I), axis=1, keepdims=True)
    id2_ref[0, 0, :] = i2[:, 0]


def _top2_indices(qraw, kn):
    id2 = pl.pallas_call(
        _corr_body,
        grid=(NQT,),
        in_specs=[
            pl.BlockSpec((QT, CKP), lambda i: (i, 0)),
            pl.BlockSpec((CKP, LP), lambda i: (0, 0)),
        ],
        out_specs=pl.BlockSpec((1, 1, QT), lambda i: (i, 0, 0)),
        out_shape=jax.ShapeDtypeStruct((NQT, 1, QT), jnp.int32),
    )(qraw, kn)
    return id2.reshape(LP)


def _sc_gather(table, idx):
    try:
        info = plsc.get_sparse_core_info()
        nc, ns = info.num_cores, info.num_subcores
    except Exception:
        nc, ns = 2, 16
    nw = nc * ns
    bpw = LP // nw
    nch = 2
    gr = bpw // nch

    @functools.partial(
        pl.kernel,
        out_type=jax.ShapeDtypeStruct((LP, CKP), jnp.float32),
        mesh=plsc.VectorSubcoreMesh(core_axis_name="c", subcore_axis_name="s"),
        scratch_types=[
            pltpu.VMEM((gr,), jnp.int32),
            pltpu.VMEM((gr, CKP), jnp.float32),
            pltpu.SemaphoreType.DMA,
        ],
    )
    def gather_kernel(table_hbm, idx_hbm, out_hbm, idx_v, rows_v, sem):
        wid = lax.axis_index("s") * nc + lax.axis_index("c")
        base = wid * bpw
        for ch in range(nch):
            b = base + ch * gr
            pltpu.sync_copy(idx_hbm.at[pl.ds(b, gr)], idx_v)
            pltpu.async_copy(table_hbm.at[idx_v], rows_v, sem).wait()
            pltpu.sync_copy(rows_v, out_hbm.at[pl.ds(b, gr)])

    return gather_kernel(table, idx)


def _c1_body(x_ref, t_ref, wa_ref, wb_ref, b_ref, a_ref, o_ref):
    i = pl.program_id(0)

    @pl.when((i >= 1) & (i <= 28))
    def _():
        y = jax.lax.dot_general(
            x_ref[...], wa_ref[...], (((0,), (0,)), ((), ())),
            preferred_element_type=jnp.float32)
        y = y + jax.lax.dot_general(
            t_ref[...], wb_ref[...], (((1,), (0,)), ((), ())),
            preferred_element_type=jnp.float32)
        y = y + b_ref[...]
        a = a_ref[0, 0]
        y = jnp.where(y >= 0, y, a * y)
        y3 = y.reshape(C1RB, W, C)
        o_ref[...] = jnp.pad(y3, ((0, 0), (1, WPAD - W - 1), (0, 0))).reshape(
            C1BLK, C)

    @pl.when((i == 0) | (i >= 29))
    def _():
        o_ref[...] = jnp.zeros((C1BLK, C), jnp.float32)


def _conv1(x_cm, t_pix, wa, wb, b1, a):
    return pl.pallas_call(
        _c1_body,
        grid=(C1N,),
        in_specs=[
            pl.BlockSpec((C, C1RB * W), lambda i: (0, jnp.clip(i - 1, 0, 27))),
            pl.BlockSpec((C1RB * W, C), lambda i: (jnp.clip(i - 1, 0, 27), 0)),
            pl.BlockSpec((C, C), lambda i: (0, 0)),
            pl.BlockSpec((C, C), lambda i: (0, 0)),
            pl.BlockSpec((1, C), lambda i: (0, 0)),
            pl.BlockSpec((1, 1), lambda i: (0, 0)),
        ],
        out_specs=pl.BlockSpec((C1BLK, C), lambda i: (i, 0)),
        out_shape=jax.ShapeDtypeStruct((HROWS, C), jnp.float32),
    )(x_cm, t_pix, wa, wb, b1, a)


def _c2_body(ha_ref, hb_ref, w_ref, b_ref, y_ref):
    acc = jnp.zeros((C2T, C), jnp.float32)
    for si, off in enumerate(OFFS):
        hs = jnp.concatenate([ha_ref[off:, :], hb_ref[:off, :]], axis=0)
        acc = acc + jax.lax.dot_general(
            hs, w_ref[si], (((1,), (0,)), ((), ())),
            preferred_element_type=jnp.float32)
    y_ref[...] = acc + b_ref[...]


def _conv2(hflat, w2s, b2):
    return pl.pallas_call(
        _c2_body,
        grid=(C2N,),
        in_specs=[
            pl.BlockSpec((C2T, C), lambda i: (i, 0)),
            pl.BlockSpec((C2T, C), lambda i: (i + 1, 0)),
            pl.BlockSpec((9, C, C), lambda i: (0, 0, 0)),
            pl.BlockSpec((1, C), lambda i: (0, 0)),
        ],
        out_specs=pl.BlockSpec((C2T, C), lambda i: (i, 0)),
        out_shape=jax.ShapeDtypeStruct((C2N * C2T, C), jnp.float32),
    )(hflat, hflat, w2s, b2)


def kernel(x8, x4, W1, b1, prelu_a, W2, b2):
    f32 = jnp.float32
    x4i = x4[0].astype(f32)
    x8i = x8[0].astype(f32)

    x4p = jnp.pad(x4i, ((0, 0), (1, 1), (1, 1)))[:, :225, :225]
    x8p = jnp.pad(x8i, ((0, 0), (1, 1), (1, 1)))[:, :225, :225]

    k5 = x4p.reshape(C, LH, 3, LH, 3)
    kraw = k5.transpose(0, 2, 4, 1, 3).reshape(1, CK, L)
    knn = jnp.sqrt(jnp.sum(kraw * kraw, axis=1, keepdims=True))
    kn = (kraw / jnp.maximum(knn, 1e-12))[0]
    kn = jnp.pad(kn, ((0, CKP - CK), (0, LP - L)))

    q5 = x8p.reshape(C, LH, 3, LH, 3)
    qraw = q5.transpose(1, 3, 0, 2, 4).reshape(1, L, CK)
    qnn = jnp.sqrt(jnp.sum(qraw * qraw, axis=2, keepdims=True))
    qn = (qraw / jnp.maximum(qnn, 1e-12))[0]
    qn = jnp.pad(qn, ((0, LP - L), (0, CKP - CK)))

    id2 = _top2_indices(qn, kn)

    xt = x4p.transpose(1, 2, 0)
    table = xt.reshape(LH, 3, LH, 3, C).transpose(0, 2, 1, 3, 4).reshape(L, CK)
    table = jnp.pad(table, ((0, LP - L), (0, CKP - CK)))

    tr = _sc_gather(table, id2)

    t225 = tr[:L, :CK].reshape(LH, LH, 3, 3, C).transpose(0, 2, 1, 3, 4)
    t225 = t225.reshape(225, 225, C)
    t_pix = t225[1:225, 1:225, :].reshape(NPIX, C)
    x4_cm = x4i.reshape(C, NPIX)

    w1t = W1[:, :, 0, 0].T.astype(f32)
    hflat = _conv1(x4_cm, t_pix, w1t[:C], w1t[C:], b1.reshape(1, C),
                   prelu_a.reshape(1, 1))

    w2s = W2.transpose(2, 3, 1, 0).reshape(9, C, C)
    y = _conv2(hflat, w2s, b2.reshape(1, C))

    out = y.reshape(H, WPAD, C)[:, :W, :]
    return out.transpose(2, 0, 1)[None].astype(x8.dtype)
